# Initial kernel scaffold; baseline (speedup 1.0000x reference)
import jax
import jax.numpy as jnp
from jax.experimental import pallas as pl


def kernel(x):
    raise NotImplementedError("write your pallas kernel here")


# speedup vs baseline: 1.2542x; 1.2542x over previous
import functools

import jax
import jax.numpy as jnp
from jax import lax
from jax.experimental import pallas as pl
from jax.experimental.pallas import tpu as pltpu
from jax.experimental.pallas import tpu_sc as plsc

_B, _S, _W = 4, 2048, 4096
_N = _B * _S * _W
_NW = 32
_PER_W = _N // _NW
_C = 16384
_NCHUNK = _PER_W // _C
_L = 16


@functools.partial(
    pl.kernel,
    out_type=jax.ShapeDtypeStruct((2 * _N,), jnp.float32),
    mesh=plsc.VectorSubcoreMesh(core_axis_name="c", subcore_axis_name="s"),
    scratch_types=[
        pltpu.VMEM((_C,), jnp.float32),
        pltpu.VMEM((2 * _C,), jnp.float32),
    ],
    compiler_params=pltpu.CompilerParams(needs_layout_passes=False),
)
def _interp_sc(x_hbm, y_hbm, in_v, out_v):
    wid = lax.axis_index("s") * 2 + lax.axis_index("c")
    base = wid * _PER_W

    zeros = jnp.zeros((_L,), jnp.float32)

    def zbody(i, carry):
        out_v[pl.ds(i * _L, _L)] = zeros
        return carry

    lax.fori_loop(0, (2 * _C) // _L, zbody, 0)

    evens = lax.iota(jnp.int32, _L) * 2

    def ibody(i, carry):
        xv = in_v[pl.ds(i * _L, _L)]
        plsc.store_scatter(out_v, [evens + i * (2 * _L)], xv)
        return carry

    def cbody(c, carry):
        off = base + c * _C
        pltpu.sync_copy(x_hbm.at[pl.ds(off, _C)], in_v)
        lax.fori_loop(0, _C // _L, ibody, 0)
        pltpu.sync_copy(out_v, y_hbm.at[pl.ds(2 * off, 2 * _C)])
        return carry

    lax.fori_loop(0, _NCHUNK, cbody, 0)


def kernel(x):
    y = _interp_sc(x.reshape(_N))
    return y.reshape(_B, _S, 2 * _W)


# speedup vs baseline: 1.5882x; 1.2663x over previous
import functools

import jax
import jax.numpy as jnp
from jax import lax
from jax.experimental import pallas as pl
from jax.experimental.pallas import tpu as pltpu
from jax.experimental.pallas import tpu_sc as plsc

_B, _S, _W = 4, 2048, 4096
_N = _B * _S * _W
_NW = 32
_PER_W = _N // _NW
_C = 16384
_NCHUNK = _PER_W // _C
_L = 16


@functools.partial(
    pl.kernel,
    out_type=jax.ShapeDtypeStruct((2 * _N,), jnp.float32),
    mesh=plsc.VectorSubcoreMesh(core_axis_name="c", subcore_axis_name="s"),
    scratch_types=[
        pltpu.VMEM((_C,), jnp.float32),
        pltpu.VMEM((_C,), jnp.float32),
        pltpu.VMEM((2 * _C,), jnp.float32),
        pltpu.VMEM((2 * _C,), jnp.float32),
        pltpu.SemaphoreType.DMA,
        pltpu.SemaphoreType.DMA,
        pltpu.SemaphoreType.DMA,
        pltpu.SemaphoreType.DMA,
    ],
    compiler_params=pltpu.CompilerParams(needs_layout_passes=False),
)
def _interp_sc(x_hbm, y_hbm, in0, in1, out0, out1, sg0, sg1, ss0, ss1):
    wid = lax.axis_index("s") * 2 + lax.axis_index("c")
    base = wid * _PER_W

    zeros = jnp.zeros((_L,), jnp.float32)

    def zbody(i, carry):
        out0[pl.ds(i * _L, _L)] = zeros
        out1[pl.ds(i * _L, _L)] = zeros
        return carry

    lax.fori_loop(0, (2 * _C) // _L, zbody, 0)

    evens = lax.iota(jnp.int32, _L) * 2

    def make_ibody(in_v, out_v):
        def ibody(i, carry):
            xv = in_v[pl.ds(i * _L, _L)]
            plsc.store_scatter(out_v, [evens + i * (2 * _L)], xv)
            return carry
        return ibody

    bufs = ((in0, out0, sg0, ss0), (in1, out1, sg1, ss1))

    pltpu.async_copy(x_hbm.at[pl.ds(base, _C)], in0, sg0)
    pltpu.async_copy(x_hbm.at[pl.ds(base + _C, _C)], in1, sg1)

    def cbody(c2, carry):
        for b, (in_v, out_v, sg, ss) in enumerate(bufs):
            cc = c2 * 2 + b
            off = base + cc * _C
            pltpu.make_async_copy(x_hbm.at[pl.ds(off, _C)], in_v, sg).wait()
            @pl.when(cc >= 2)
            def _():
                pltpu.make_async_copy(
                    out_v, y_hbm.at[pl.ds(2 * off, 2 * _C)], ss).wait()
            lax.fori_loop(0, _C // _L, make_ibody(in_v, out_v), 0)
            pltpu.async_copy(out_v, y_hbm.at[pl.ds(2 * off, 2 * _C)], ss)
            @pl.when(cc + 2 < _NCHUNK)
            def _():
                pltpu.async_copy(
                    x_hbm.at[pl.ds(off + 2 * _C, _C)], in_v, sg)
        return carry

    lax.fori_loop(0, _NCHUNK // 2, cbody, 0)

    tail0 = base + (_NCHUNK - 2) * _C
    tail1 = base + (_NCHUNK - 1) * _C
    pltpu.make_async_copy(out0, y_hbm.at[pl.ds(2 * tail0, 2 * _C)], ss0).wait()
    pltpu.make_async_copy(out1, y_hbm.at[pl.ds(2 * tail1, 2 * _C)], ss1).wait()


def kernel(x):
    y = _interp_sc(x.reshape(_N))
    return y.reshape(_B, _S, 2 * _W)


# speedup vs baseline: 2.4531x; 1.5445x over previous
import functools

import jax
import jax.numpy as jnp
from jax import lax
from jax.experimental import pallas as pl
from jax.experimental.pallas import tpu as pltpu
from jax.experimental.pallas import tpu_sc as plsc

_B, _S, _W = 4, 2048, 4096
_R = _B * _S
_NW = 32
_RPW = _R // _NW
_CR = 8
_CC = 2048
_L = 16


@functools.partial(
    pl.kernel,
    out_type=jax.ShapeDtypeStruct((_R, 2 * _W), jnp.float32),
    mesh=plsc.VectorSubcoreMesh(core_axis_name="c", subcore_axis_name="s"),
    scratch_types=[
        pltpu.VMEM((_CR, _CC), jnp.float32),
        pltpu.VMEM((_CR, 2 * _CC), jnp.float32),
    ],
    compiler_params=pltpu.CompilerParams(
        needs_layout_passes=False, use_tc_tiling_on_sc=True),
)
def _interp_sc(x_hbm, y_hbm, in_v, out_v):
    wid = lax.axis_index("s") * 2 + lax.axis_index("c")
    row0 = wid * _RPW

    zeros = jnp.zeros((_L,), jnp.float32)

    def zbody(i, carry):
        for s in range(_CR):
            out_v[s, pl.ds(i * _L, _L)] = zeros
        return carry

    lax.fori_loop(0, (2 * _CC) // _L, zbody, 0)

    evens = lax.iota(jnp.int32, _L) * 2

    def ibody(i, carry):
        for s in range(_CR):
            xv = in_v[s, pl.ds(i * _L, _L)]
            rows = jnp.full((_L,), s, jnp.int32)
            plsc.store_scatter(out_v, [rows, evens + i * (2 * _L)], xv)
        return carry

    def cbody(c, carry):
        r = row0 + (c // (_W // _CC)) * _CR
        col = (c % (_W // _CC)) * _CC
        pltpu.sync_copy(x_hbm.at[pl.ds(r, _CR), pl.ds(col, _CC)], in_v)
        lax.fori_loop(0, _CC // _L, ibody, 0)
        pltpu.sync_copy(
            out_v, y_hbm.at[pl.ds(r, _CR), pl.ds(2 * col, 2 * _CC)])
        return carry

    lax.fori_loop(0, (_RPW // _CR) * (_W // _CC), cbody, 0)


def kernel(x):
    y = _interp_sc(x.reshape(_R, _W))
    return y.reshape(_B, _S, 2 * _W)


# speedup vs baseline: 4.1323x; 1.6846x over previous
import functools

import jax
import jax.numpy as jnp
from jax import lax
from jax.experimental import pallas as pl
from jax.experimental.pallas import tpu as pltpu
from jax.experimental.pallas import tpu_sc as plsc

_B, _S, _W = 4, 2048, 4096
_R = _B * _S
_NW = 32
_RPW = _R // _NW
_CR = 8
_CC = 2048
_L = 16


@functools.partial(
    pl.kernel,
    out_type=jax.ShapeDtypeStruct((_R, 2 * _W), jnp.float32),
    mesh=plsc.VectorSubcoreMesh(core_axis_name="c", subcore_axis_name="s"),
    scratch_types=[
        pltpu.VMEM((_CR, _CC), jnp.float32),
        pltpu.VMEM((_CR, _CC), jnp.float32),
        pltpu.VMEM((_CR, 2 * _CC), jnp.float32),
        pltpu.VMEM((_CR, 2 * _CC), jnp.float32),
        pltpu.SemaphoreType.DMA,
        pltpu.SemaphoreType.DMA,
        pltpu.SemaphoreType.DMA,
        pltpu.SemaphoreType.DMA,
    ],
    compiler_params=pltpu.CompilerParams(
        needs_layout_passes=False, use_tc_tiling_on_sc=True),
)
def _interp_sc(x_hbm, y_hbm, in0, in1, out0, out1, sg0, sg1, ss0, ss1):
    wid = lax.axis_index("s") * 2 + lax.axis_index("c")
    row0 = wid * _RPW
    _CPR = _W // _CC
    _NCHUNK = (_RPW // _CR) * _CPR

    zeros = jnp.zeros((_L,), jnp.float32)

    def zbody(i, carry):
        for s in range(_CR):
            out0[s, pl.ds(i * _L, _L)] = zeros
            out1[s, pl.ds(i * _L, _L)] = zeros
        return carry

    lax.fori_loop(0, (2 * _CC) // _L, zbody, 0)

    evens = lax.iota(jnp.int32, _L) * 2
    rowsel = [jnp.full((_L,), s, jnp.int32) for s in range(_CR)]

    def src_slice(c):
        r = row0 + (c // _CPR) * _CR
        col = (c % _CPR) * _CC
        return x_hbm.at[pl.ds(r, _CR), pl.ds(col, _CC)]

    def dst_slice(c):
        r = row0 + (c // _CPR) * _CR
        col = (c % _CPR) * _CC
        return y_hbm.at[pl.ds(r, _CR), pl.ds(2 * col, 2 * _CC)]

    def make_ibody(in_v, out_v):
        def ibody(i, carry):
            for s in range(_CR):
                xv = in_v[s, pl.ds(i * _L, _L)]
                plsc.store_scatter(
                    out_v, [rowsel[s], evens + i * (2 * _L)], xv)
            return carry
        return ibody

    bufs = ((in0, out0, sg0, ss0), (in1, out1, sg1, ss1))

    pltpu.async_copy(src_slice(0), in0, sg0)
    pltpu.async_copy(src_slice(1), in1, sg1)

    def cbody(c2, carry):
        for b, (in_v, out_v, sg, ss) in enumerate(bufs):
            cc = c2 * 2 + b
            pltpu.make_async_copy(src_slice(cc), in_v, sg).wait()
            @pl.when(cc >= 2)
            def _():
                pltpu.make_async_copy(out_v, dst_slice(cc), ss).wait()
            lax.fori_loop(0, _CC // _L, make_ibody(in_v, out_v), 0)
            pltpu.async_copy(out_v, dst_slice(cc), ss)
            @pl.when(cc + 2 < _NCHUNK)
            def _():
                pltpu.async_copy(src_slice(cc + 2), in_v, sg)
        return carry

    lax.fori_loop(0, _NCHUNK // 2, cbody, 0)

    pltpu.make_async_copy(out0, dst_slice(_NCHUNK - 2), ss0).wait()
    pltpu.make_async_copy(out1, dst_slice(_NCHUNK - 1), ss1).wait()


def kernel(x):
    y = _interp_sc(x.reshape(_R, _W))
    return y.reshape(_B, _S, 2 * _W)


# speedup vs baseline: 4.1527x; 1.0049x over previous
import functools

import jax
import jax.numpy as jnp
from jax import lax
from jax.experimental import pallas as pl
from jax.experimental.pallas import tpu as pltpu
from jax.experimental.pallas import tpu_sc as plsc

_B, _S, _W = 4, 2048, 4096
_R = _B * _S
_NW = 32
_RPW = _R // _NW
_CR = 8
_CC = 2048
_L = 16


@functools.partial(
    pl.kernel,
    out_type=jax.ShapeDtypeStruct((_R, 2 * _W), jnp.float32),
    mesh=plsc.VectorSubcoreMesh(core_axis_name="c", subcore_axis_name="s"),
    scratch_types=[
        pltpu.VMEM((_CR, _CC), jnp.float32),
        pltpu.VMEM((_CR, _CC), jnp.float32),
        pltpu.VMEM((_CR, 2 * _CC), jnp.float32),
        pltpu.VMEM((_CR, 2 * _CC), jnp.float32),
        pltpu.SemaphoreType.DMA,
        pltpu.SemaphoreType.DMA,
        pltpu.SemaphoreType.DMA,
        pltpu.SemaphoreType.DMA,
    ],
    compiler_params=pltpu.CompilerParams(
        needs_layout_passes=False, use_tc_tiling_on_sc=True),
)
def _interp_sc(x_hbm, y_hbm, in0, in1, out0, out1, sg0, sg1, ss0, ss1):
    wid = lax.axis_index("s") * 2 + lax.axis_index("c")
    row0 = wid * _RPW
    _CPR = _W // _CC
    _NCHUNK = (_RPW // _CR) * _CPR

    evens = lax.iota(jnp.int32, _L) * 2
    rowsel = [jnp.full((_L,), s, jnp.int32) for s in range(_CR)]

    def src_slice(c):
        r = row0 + (c // _CPR) * _CR
        col = (c % _CPR) * _CC
        return x_hbm.at[pl.ds(r, _CR), pl.ds(col, _CC)]

    def dst_slice(c):
        r = row0 + (c // _CPR) * _CR
        col = (c % _CPR) * _CC
        return y_hbm.at[pl.ds(r, _CR), pl.ds(2 * col, 2 * _CC)]

    def make_ibody(in_v, out_v):
        def ibody(i, carry):
            for s in range(_CR):
                xv = in_v[s, pl.ds(i * _L, _L)]
                plsc.store_scatter(
                    out_v, [rowsel[s], evens + i * (2 * _L)], xv)
            return carry
        return ibody

    bufs = ((in0, out0, sg0, ss0), (in1, out1, sg1, ss1))

    pltpu.async_copy(src_slice(0), in0, sg0)
    pltpu.async_copy(src_slice(1), in1, sg1)

    zeros = jnp.zeros((_L,), jnp.float32)

    def zbody(i, carry):
        for s in range(_CR):
            out0[s, pl.ds(i * _L, _L)] = zeros
            out1[s, pl.ds(i * _L, _L)] = zeros
        return carry

    lax.fori_loop(0, (2 * _CC) // _L, zbody, 0)

    def cbody(c2, carry):
        for b, (in_v, out_v, sg, ss) in enumerate(bufs):
            cc = c2 * 2 + b
            pltpu.make_async_copy(src_slice(cc), in_v, sg).wait()
            @pl.when(cc >= 2)
            def _():
                pltpu.make_async_copy(out_v, dst_slice(cc), ss).wait()
            lax.fori_loop(0, _CC // _L, make_ibody(in_v, out_v), 0)
            pltpu.async_copy(out_v, dst_slice(cc), ss)
            @pl.when(cc + 2 < _NCHUNK)
            def _():
                pltpu.async_copy(src_slice(cc + 2), in_v, sg)
        return carry

    lax.fori_loop(0, _NCHUNK // 2, cbody, 0)

    pltpu.make_async_copy(out0, dst_slice(_NCHUNK - 2), ss0).wait()
    pltpu.make_async_copy(out1, dst_slice(_NCHUNK - 1), ss1).wait()


def kernel(x):
    y = _interp_sc(x.reshape(_R, _W))
    return y.reshape(_B, _S, 2 * _W)
